# baseline (speedup 1.0000x reference)
import functools

import jax
import jax.numpy as jnp
from jax import lax
from jax.experimental import pallas as pl
from jax.experimental.pallas import tpu as pltpu
from jax.experimental.pallas import tpu_sc as plsc

N = 10000
E = 160000
EPAD = 163840
NT = 16
NCORE = 2
CHUNK = 128
ROWS1 = 80
ROWS2 = 40
F = 128
NFC = 4
NPAD = 10240
DEN = 10240
NBUF = 4864
MBLK = 1000
NMB = 10

ZONES = ((0, NBUF, 304, (128, 128, 48)),
         (NBUF, NBUF, 304, (128, 128, 48)),
         (2 * NBUF, 512, 32, (32,)))

_f32 = jnp.float32


def _bcast_lane(v16, lane):
    idx = jnp.full((16, 1), lane, dtype=jnp.int32)
    dnums = lax.GatherDimensionNumbers(
        offset_dims=(), collapsed_slice_dims=(0,), start_index_map=(0,))
    return lax.gather(v16, idx, dnums, (1,),
                      mode=lax.GatherScatterMode.PROMISE_IN_BOUNDS)


def _p1_body(x_ref, w_ref, al_ref, ar_ref, h0, h1, h2, h3, el_ref, er_ref):
    h = jnp.dot(x_ref[...], w_ref[...], preferred_element_type=_f32)
    h0[...] = h[:, 0:128]
    h1[...] = h[:, 128:256]
    h2[...] = h[:, 256:384]
    h3[...] = h[:, 384:512]
    el_ref[...] = jnp.dot(h, al_ref[...], preferred_element_type=_f32)
    er_ref[...] = jnp.dot(h, ar_ref[...], preferred_element_type=_f32)


_p1 = pl.pallas_call(
    _p1_body,
    grid=(NMB,),
    in_specs=[
        pl.BlockSpec((MBLK, 256), lambda m: (m, 0)),
        pl.BlockSpec((256, 512), lambda m: (0, 0)),
        pl.BlockSpec((512, 1), lambda m: (0, 0)),
        pl.BlockSpec((512, 1), lambda m: (0, 0)),
    ],
    out_specs=[pl.BlockSpec((MBLK, F), lambda m: (m, 0))] * 4
    + [pl.BlockSpec((MBLK, 1), lambda m: (m, 0))] * 2,
    out_shape=[jax.ShapeDtypeStruct((N, F), _f32)] * 4
    + [jax.ShapeDtypeStruct((N, 1), _f32)] * 2,
)


def _p3_body(pd_ref, b_ref, w_ref, al_ref, ar_ref,
             h0, h1, h2, h3, el_ref, er_ref, acc_ref):
    c = pl.program_id(1)
    xin = jnp.maximum(pd_ref[0, 0] + pd_ref[1, 0] + b_ref[...], 0.0)
    part = jnp.dot(xin, w_ref[...], preferred_element_type=_f32)

    @pl.when(c == 0)
    def _():
        acc_ref[...] = part

    @pl.when(c > 0)
    def _():
        acc_ref[...] = acc_ref[...] + part

    @pl.when(c == NFC - 1)
    def _():
        h = acc_ref[...]
        h0[...] = h[:, 0:128]
        h1[...] = h[:, 128:256]
        h2[...] = h[:, 256:384]
        h3[...] = h[:, 384:512]
        el_ref[...] = jnp.dot(h, al_ref[...], preferred_element_type=_f32)
        er_ref[...] = jnp.dot(h, ar_ref[...], preferred_element_type=_f32)


_p3 = pl.pallas_call(
    _p3_body,
    grid=(NMB, NFC),
    in_specs=[
        pl.BlockSpec((2, 1, MBLK, F), lambda m, c: (0, c, m, 0)),
        pl.BlockSpec((1, F), lambda m, c: (0, c)),
        pl.BlockSpec((F, 512), lambda m, c: (c, 0)),
        pl.BlockSpec((512, 1), lambda m, c: (0, 0)),
        pl.BlockSpec((512, 1), lambda m, c: (0, 0)),
    ],
    out_specs=[pl.BlockSpec((MBLK, F), lambda m, c: (m, 0))] * 4
    + [pl.BlockSpec((MBLK, 1), lambda m, c: (m, 0))] * 2,
    out_shape=[jax.ShapeDtypeStruct((N, F), _f32)] * 4
    + [jax.ShapeDtypeStruct((N, 1), _f32)] * 2,
    scratch_shapes=[pltpu.VMEM((MBLK, 512), _f32)],
    compiler_params=pltpu.CompilerParams(
        dimension_semantics=("parallel", "arbitrary")),
)


def _p5_body(pd_ref, b_ref, f1w_ref, f1b_ref, f2w_ref, f2b_ref,
             out_ref, acc_ref):
    m = pl.program_id(0)
    xin = jnp.maximum(pd_ref[0] + pd_ref[1] + b_ref[...][:, None, :], 0.0)
    cs = jnp.sum(xin, axis=1)

    @pl.when(m == 0)
    def _():
        acc_ref[...] = cs

    @pl.when(m > 0)
    def _():
        acc_ref[...] = acc_ref[...] + cs

    @pl.when(m == NMB - 1)
    def _():
        hh = f1b_ref[...]
        for c in range(NFC):
            hg_c = acc_ref[c:c + 1, :] * _f32(1.0 / N)
            hh = hh + jnp.dot(hg_c, f1w_ref[c * F:(c + 1) * F, :],
                              preferred_element_type=_f32)
        hh = jnp.maximum(hh, 0.0)
        out_ref[...] = jnp.dot(hh, f2w_ref[...],
                               preferred_element_type=_f32) + f2b_ref[...]


_p5 = pl.pallas_call(
    _p5_body,
    grid=(NMB,),
    in_specs=[
        pl.BlockSpec((2, NFC, MBLK, F), lambda m: (0, 0, m, 0)),
        pl.BlockSpec((NFC, F), lambda m: (0, 0)),
        pl.BlockSpec((512, 512), lambda m: (0, 0)),
        pl.BlockSpec((1, 512), lambda m: (0, 0)),
        pl.BlockSpec((512, F), lambda m: (0, 0)),
        pl.BlockSpec((1, F), lambda m: (0, 0)),
    ],
    out_specs=[pl.BlockSpec((1, F), lambda m: (0, 0))],
    out_shape=[jax.ShapeDtypeStruct((1, F), _f32)],
    scratch_shapes=[pltpu.VMEM((NFC, F), _f32)],
    compiler_params=pltpu.CompilerParams(
        dimension_semantics=("arbitrary",)),
)


def _sc_edge_body(h0, h1, h2, h3, el_h, er_h, src_h, dst_h, out_pd,
                  el_v, erden_v, src_v, dst_v, ex_v, zbuf, zd_v,
                  gbuf0, gbuf1, dstmod_v, den_sh, out_sh,
                  sem, semg0, semg1, sems0, sems1):
    core = lax.axis_index("c")
    tid = lax.axis_index("s")
    hcs = (h0, h1, h2, h3)
    gbufs = (gbuf0, gbuf1)
    semgs = (semg0, semg1)
    semss = (sems0, sems1)

    def _zb(i, carry):
        for q in range(8):
            zbuf[i, pl.ds(q * 16, 16)] = jnp.zeros((16,), _f32)
        return carry

    lax.fori_loop(0, 32, _zb, 0)

    def _zd(i, carry):
        zd_v[pl.ds(i * 16, 16)] = jnp.zeros((16,), _f32)
        return carry

    lax.fori_loop(0, 40, _zd, 0)

    pltpu.sync_copy(el_h, el_v)
    pltpu.sync_copy(er_h, erden_v.at[pl.ds(0, N)])
    pltpu.sync_copy(src_h.at[tid], src_v)
    pltpu.sync_copy(dst_h.at[tid], dst_v)
    pltpu.sync_copy(zd_v, den_sh.at[pl.ds(tid * 640, 640)])
    plsc.subcore_barrier()

    def _s1(j, carry):
        for q in range(8):
            sl = pl.ds(q * 16, 16)
            sv = src_v[j, sl]
            dv = dst_v[j, sl]
            es = plsc.load_gather(el_v, [sv])
            ed = plsc.load_gather(erden_v, [dv])
            z = es + ed
            e = jnp.where(z >= 0.0, z, z * _f32(0.2))
            ex = jnp.exp(e)
            gid = tid * 10240 + j * 128 + q * 16 + lax.iota(jnp.int32, 16)
            ex = jnp.where(gid < E, ex, _f32(0.0))
            ex_v[j, sl] = ex
        return carry

    lax.fori_loop(0, ROWS1, _s1, 0)

    for blk in range(ROWS1 // 16):
        def _fire(i, carry, blk=blk):
            j = blk * 16 + i
            pltpu.async_copy(ex_v.at[j], den_sh.at[dst_v.at[j]], sem,
                             add=True)
            return carry

        lax.fori_loop(0, 16, _fire, 0)

        def _drain(i, carry, blk=blk):
            j = blk * 16 + i
            pltpu.make_async_copy(ex_v.at[j], den_sh.at[dst_v.at[j]],
                                  sem).wait()
            return carry

        lax.fori_loop(0, 16, _drain, 0)

    plsc.subcore_barrier()
    pltpu.sync_copy(den_sh, erden_v)

    def _s1b(j, carry):
        jabs = core * ROWS2 + j
        for q in range(8):
            sl = pl.ds(q * 16, 16)
            ex = ex_v[jabs, sl]
            dv = dst_v[jabs, sl]
            d = plsc.load_gather(erden_v, [dv])
            ex_v[jabs, sl] = ex / jnp.maximum(d, _f32(1e-30))
        return carry

    lax.fori_loop(0, ROWS2, _s1b, 0)

    for cf in range(NFC):
        hc = hcs[cf]
        for base, nbuf, rpt, wchunks in ZONES:
            plsc.subcore_barrier()
            off = 0
            for ln in wchunks:
                while ln > 0:
                    zl = min(ln, 32)
                    pltpu.sync_copy(zbuf.at[pl.ds(0, zl)],
                                    out_sh.at[pl.ds(tid * rpt + off, zl)])
                    off += zl
                    ln -= zl
            plsc.subcore_barrier()

            pltpu.async_copy(hc.at[src_v.at[core * ROWS2]], gbufs[0],
                             semgs[0])

            def _s2(jj, carry, hc=hc, base=base, nbuf=nbuf):
                for u in range(2):
                    j = jj * 2 + u
                    un = u ^ 1
                    jabs = core * ROWS2 + j

                    @pl.when((j >= 1) & (j + 1 < ROWS2))
                    def _(un=un):
                        pltpu.make_async_copy(
                            gbufs[un], out_sh.at[dstmod_v.at[un]],
                            semss[un]).wait()

                    @pl.when(j + 1 < ROWS2)
                    def _(jabs=jabs, un=un):
                        pltpu.async_copy(hc.at[src_v.at[jabs + 1]],
                                         gbufs[un], semgs[un])

                    gb = gbufs[u]
                    pltpu.make_async_copy(
                        hc.at[src_v.at[jabs]], gb, semgs[u]).wait()

                    def _mul(q, c2, jabs=jabs, u=u, gb=gb):
                        sl = pl.ds(q * 16, 16)
                        av = ex_v[jabs, sl]
                        dv = dst_v[jabs, sl]
                        local = dv - base
                        valid = (local >= 0) & (local < nbuf)
                        av = jnp.where(valid, av, _f32(0.0))
                        idxc = jnp.clip(local, 0, nbuf - 1)
                        dstmod_v[u, sl] = idxc
                        for r2 in range(16):
                            a = _bcast_lane(av, r2)
                            r = q * 16 + r2
                            for k2 in range(8):
                                slk = pl.ds(k2 * 16, 16)
                                gb[r, slk] = gb[r, slk] * a
                        return c2

                    lax.fori_loop(0, 8, _mul, 0)
                    pltpu.async_copy(gb, out_sh.at[dstmod_v.at[u]],
                                     semss[u], add=True)
                return carry

            lax.fori_loop(0, ROWS2 // 2, _s2, 0)
            for u in (0, 1):
                pltpu.make_async_copy(gbufs[u], out_sh.at[dstmod_v.at[u]],
                                      semss[u]).wait()
            plsc.subcore_barrier()
            off = 0
            for ln in wchunks:
                r0 = tid * rpt + off
                pltpu.sync_copy(out_sh.at[pl.ds(r0, ln)],
                                out_pd.at[core, cf, pl.ds(base + r0, ln)])
                off += ln


_sc_edge = functools.partial(
    pl.kernel,
    mesh=plsc.VectorSubcoreMesh(core_axis_name="c", subcore_axis_name="s"),
    out_type=jax.ShapeDtypeStruct((NCORE, NFC, NPAD, F), _f32),
    scratch_types=[
        pltpu.VMEM((N,), _f32),
        pltpu.VMEM((DEN,), _f32),
        pltpu.VMEM((ROWS1, CHUNK), jnp.int32),
        pltpu.VMEM((ROWS1, CHUNK), jnp.int32),
        pltpu.VMEM((ROWS1, CHUNK), _f32),
        pltpu.VMEM((32, F), _f32),
        pltpu.VMEM((640,), _f32),
        pltpu.VMEM((CHUNK, F), _f32),
        pltpu.VMEM((CHUNK, F), _f32),
        pltpu.VMEM((2, CHUNK), jnp.int32),
        pltpu.VMEM_SHARED((DEN,), _f32),
        pltpu.VMEM_SHARED((NBUF, F), _f32),
        pltpu.SemaphoreType.DMA,
        pltpu.SemaphoreType.DMA,
        pltpu.SemaphoreType.DMA,
        pltpu.SemaphoreType.DMA,
        pltpu.SemaphoreType.DMA,
    ],
    compiler_params=pltpu.CompilerParams(needs_layout_passes=False),
)(_sc_edge_body)


def kernel(x, edge_index, W1, al1, ar1, b1, W2, al2, ar2, b2,
           fc1_W, fc1_b, fc2_W, fc2_b):
    ei = edge_index.astype(jnp.int32)
    src = jnp.pad(ei[0], (0, EPAD - E)).reshape(NT, ROWS1, CHUNK)
    dst = jnp.pad(ei[1], (0, EPAD - E)).reshape(NT, ROWS1, CHUNK)

    h1c0, h1c1, h1c2, h1c3, el1, er1 = _p1(
        x, W1, al1.reshape(512, 1), ar1.reshape(512, 1))
    pd1 = _sc_edge(h1c0, h1c1, h1c2, h1c3,
                   el1.reshape(N), er1.reshape(N), src, dst)

    h2c0, h2c1, h2c2, h2c3, el2, er2 = _p3(
        pd1, b1.reshape(1, 512), W2, al2.reshape(512, 1), ar2.reshape(512, 1))
    pd2 = _sc_edge(h2c0, h2c1, h2c2, h2c3,
                   el2.reshape(N), er2.reshape(N), src, dst)

    fc2_Wp = jnp.pad(fc2_W, ((0, 0), (0, F - 2)))
    fc2_bp = jnp.pad(fc2_b, (0, F - 2)).reshape(1, F)
    (res,) = _p5(pd2, b2.reshape(NFC, F), fc1_W, fc1_b.reshape(1, 512),
                 fc2_Wp, fc2_bp)
    return res[0, :2]


# speedup vs baseline: 1.7517x; 1.7517x over previous
import functools

import jax
import jax.numpy as jnp
from jax import lax
from jax.experimental import pallas as pl
from jax.experimental.pallas import tpu as pltpu
from jax.experimental.pallas import tpu_sc as plsc

N = 10000
E = 160000
EPAD = 163840
NT = 16
NCORE = 2
CHUNK = 128
ROWS1 = 80
ROWS2 = 40
F = 128
NFC = 4
NPAD = 10240
DEN = 10240
NBUF = 5120
C0R = 56
MBLK = 1000
NMB = 10

ZONES = ((0, NBUF, 320, (128, 128, 64)),
         (NBUF, NBUF, 320, (128, 128, 64)))

_f32 = jnp.float32


def _bcast_lane(v16, lane):
    idx = jnp.full((16, 1), lane, dtype=jnp.int32)
    dnums = lax.GatherDimensionNumbers(
        offset_dims=(), collapsed_slice_dims=(0,), start_index_map=(0,))
    return lax.gather(v16, idx, dnums, (1,),
                      mode=lax.GatherScatterMode.PROMISE_IN_BOUNDS)


def _p1_body(x_ref, w_ref, al_ref, ar_ref, h0, h1, h2, h3, el_ref, er_ref):
    h = jnp.dot(x_ref[...], w_ref[...], preferred_element_type=_f32)
    h0[...] = h[:, 0:128]
    h1[...] = h[:, 128:256]
    h2[...] = h[:, 256:384]
    h3[...] = h[:, 384:512]
    el_ref[...] = jnp.dot(h, al_ref[...], preferred_element_type=_f32)
    er_ref[...] = jnp.dot(h, ar_ref[...], preferred_element_type=_f32)


_p1 = pl.pallas_call(
    _p1_body,
    grid=(NMB,),
    in_specs=[
        pl.BlockSpec((MBLK, 256), lambda m: (m, 0)),
        pl.BlockSpec((256, 512), lambda m: (0, 0)),
        pl.BlockSpec((512, 1), lambda m: (0, 0)),
        pl.BlockSpec((512, 1), lambda m: (0, 0)),
    ],
    out_specs=[pl.BlockSpec((MBLK, F), lambda m: (m, 0))] * 4
    + [pl.BlockSpec((MBLK, 1), lambda m: (m, 0))] * 2,
    out_shape=[jax.ShapeDtypeStruct((N, F), _f32)] * 4
    + [jax.ShapeDtypeStruct((N, 1), _f32)] * 2,
)


def _p3_body(pd_ref, b_ref, w_ref, al_ref, ar_ref,
             h0, h1, h2, h3, el_ref, er_ref, acc_ref):
    c = pl.program_id(1)
    xin = jnp.maximum(pd_ref[0, 0] + pd_ref[1, 0] + b_ref[...], 0.0)
    part = jnp.dot(xin, w_ref[...], preferred_element_type=_f32)

    @pl.when(c == 0)
    def _():
        acc_ref[...] = part

    @pl.when(c > 0)
    def _():
        acc_ref[...] = acc_ref[...] + part

    @pl.when(c == NFC - 1)
    def _():
        h = acc_ref[...]
        h0[...] = h[:, 0:128]
        h1[...] = h[:, 128:256]
        h2[...] = h[:, 256:384]
        h3[...] = h[:, 384:512]
        el_ref[...] = jnp.dot(h, al_ref[...], preferred_element_type=_f32)
        er_ref[...] = jnp.dot(h, ar_ref[...], preferred_element_type=_f32)


_p3 = pl.pallas_call(
    _p3_body,
    grid=(NMB, NFC),
    in_specs=[
        pl.BlockSpec((2, 1, MBLK, F), lambda m, c: (0, c, m, 0)),
        pl.BlockSpec((1, F), lambda m, c: (0, c)),
        pl.BlockSpec((F, 512), lambda m, c: (c, 0)),
        pl.BlockSpec((512, 1), lambda m, c: (0, 0)),
        pl.BlockSpec((512, 1), lambda m, c: (0, 0)),
    ],
    out_specs=[pl.BlockSpec((MBLK, F), lambda m, c: (m, 0))] * 4
    + [pl.BlockSpec((MBLK, 1), lambda m, c: (m, 0))] * 2,
    out_shape=[jax.ShapeDtypeStruct((N, F), _f32)] * 4
    + [jax.ShapeDtypeStruct((N, 1), _f32)] * 2,
    scratch_shapes=[pltpu.VMEM((MBLK, 512), _f32)],
    compiler_params=pltpu.CompilerParams(
        dimension_semantics=("parallel", "arbitrary")),
)


def _p5_body(pd_ref, b_ref, f1w_ref, f1b_ref, f2w_ref, f2b_ref,
             out_ref, acc_ref):
    m = pl.program_id(0)
    xin = jnp.maximum(pd_ref[0] + pd_ref[1] + b_ref[...][:, None, :], 0.0)
    cs = jnp.sum(xin, axis=1)

    @pl.when(m == 0)
    def _():
        acc_ref[...] = cs

    @pl.when(m > 0)
    def _():
        acc_ref[...] = acc_ref[...] + cs

    @pl.when(m == NMB - 1)
    def _():
        hh = f1b_ref[...]
        for c in range(NFC):
            hg_c = acc_ref[c:c + 1, :] * _f32(1.0 / N)
            hh = hh + jnp.dot(hg_c, f1w_ref[c * F:(c + 1) * F, :],
                              preferred_element_type=_f32)
        hh = jnp.maximum(hh, 0.0)
        out_ref[...] = jnp.dot(hh, f2w_ref[...],
                               preferred_element_type=_f32) + f2b_ref[...]


_p5 = pl.pallas_call(
    _p5_body,
    grid=(NMB,),
    in_specs=[
        pl.BlockSpec((2, NFC, MBLK, F), lambda m: (0, 0, m, 0)),
        pl.BlockSpec((NFC, F), lambda m: (0, 0)),
        pl.BlockSpec((512, 512), lambda m: (0, 0)),
        pl.BlockSpec((1, 512), lambda m: (0, 0)),
        pl.BlockSpec((512, F), lambda m: (0, 0)),
        pl.BlockSpec((1, F), lambda m: (0, 0)),
    ],
    out_specs=[pl.BlockSpec((1, F), lambda m: (0, 0))],
    out_shape=[jax.ShapeDtypeStruct((1, F), _f32)],
    scratch_shapes=[pltpu.VMEM((NFC, F), _f32)],
    compiler_params=pltpu.CompilerParams(
        dimension_semantics=("arbitrary",)),
)


def _sc_edge_body(h0, h1, h2, h3, el_h, er_h, src_h, dst_h, out_pd,
                  el_v, erden_v, src_v, dst_v, ex_v, zbuf, zd_v,
                  gbuf0, gbuf1, dstmod_v, den_sh, out_sh,
                  sem, semg0, semg1, sems0, sems1):
    core = lax.axis_index("c")
    tid = lax.axis_index("s")
    hcs = (h0, h1, h2, h3)
    gbufs = (gbuf0, gbuf1)
    semgs = (semg0, semg1)
    semss = (sems0, sems1)

    def _zb(i, carry):
        for q in range(8):
            zbuf[i, pl.ds(q * 16, 16)] = jnp.zeros((16,), _f32)
        return carry

    lax.fori_loop(0, 16, _zb, 0)

    def _zd(i, carry):
        zd_v[pl.ds(i * 16, 16)] = jnp.zeros((16,), _f32)
        return carry

    lax.fori_loop(0, 40, _zd, 0)

    pltpu.sync_copy(el_h, el_v)
    pltpu.sync_copy(er_h, erden_v.at[pl.ds(0, N)])
    pltpu.sync_copy(src_h.at[tid], src_v)
    pltpu.sync_copy(dst_h.at[tid], dst_v)
    pltpu.sync_copy(zd_v, den_sh.at[pl.ds(tid * 640, 640)])
    plsc.subcore_barrier()

    def _s1(j, carry):
        for q in range(8):
            sl = pl.ds(q * 16, 16)
            sv = src_v[j, sl]
            dv = dst_v[j, sl]
            es = plsc.load_gather(el_v, [sv])
            ed = plsc.load_gather(erden_v, [dv])
            z = es + ed
            e = jnp.where(z >= 0.0, z, z * _f32(0.2))
            ex = jnp.exp(e)
            gid = tid * 10240 + j * 128 + q * 16 + lax.iota(jnp.int32, 16)
            ex = jnp.where(gid < E, ex, _f32(0.0))
            ex_v[j, sl] = ex
        return carry

    lax.fori_loop(0, ROWS1, _s1, 0)

    for blk in range(ROWS1 // 16):
        def _fire(i, carry, blk=blk):
            j = blk * 16 + i
            pltpu.async_copy(ex_v.at[j], den_sh.at[dst_v.at[j]], sem,
                             add=True)
            return carry

        lax.fori_loop(0, 16, _fire, 0)

        def _drain(i, carry, blk=blk):
            j = blk * 16 + i
            pltpu.make_async_copy(ex_v.at[j], den_sh.at[dst_v.at[j]],
                                  sem).wait()
            return carry

        lax.fori_loop(0, 16, _drain, 0)

    plsc.subcore_barrier()
    pltpu.sync_copy(den_sh, erden_v)

    jlo = core * C0R
    jcnt = C0R - core * (2 * C0R - ROWS1)

    def _s1b(j, carry):
        jabs = jlo + j
        for q in range(8):
            sl = pl.ds(q * 16, 16)
            ex = ex_v[jabs, sl]
            dv = dst_v[jabs, sl]
            d = plsc.load_gather(erden_v, [dv])
            ex_v[jabs, sl] = ex / jnp.maximum(d, _f32(1e-30))
        return carry

    lax.fori_loop(0, jcnt, _s1b, 0)

    for cf in range(NFC):
        hc = hcs[cf]
        for base, nbuf, rpt, wchunks in ZONES:
            plsc.subcore_barrier()
            off = 0
            for ln in wchunks:
                while ln > 0:
                    zl = min(ln, 16)
                    pltpu.sync_copy(zbuf.at[pl.ds(0, zl)],
                                    out_sh.at[pl.ds(tid * rpt + off, zl)])
                    off += zl
                    ln -= zl
            plsc.subcore_barrier()

            pltpu.async_copy(hc.at[src_v.at[jlo]], gbufs[0],
                             semgs[0])

            def _s2(jj, carry, hc=hc, base=base, nbuf=nbuf):
                for u in range(2):
                    j = jj * 2 + u
                    un = u ^ 1
                    jabs = jlo + j

                    @pl.when((j >= 1) & (j + 1 < jcnt))
                    def _(un=un):
                        pltpu.make_async_copy(
                            gbufs[un], out_sh.at[dstmod_v.at[un]],
                            semss[un]).wait()

                    @pl.when(j + 1 < jcnt)
                    def _(jabs=jabs, un=un):
                        pltpu.async_copy(hc.at[src_v.at[jabs + 1]],
                                         gbufs[un], semgs[un])

                    gb = gbufs[u]
                    pltpu.make_async_copy(
                        hc.at[src_v.at[jabs]], gb, semgs[u]).wait()

                    def _mul(q, c2, jabs=jabs, u=u, gb=gb):
                        sl = pl.ds(q * 16, 16)
                        av = ex_v[jabs, sl]
                        dv = dst_v[jabs, sl]
                        local = dv - base
                        valid = (local >= 0) & (local < nbuf)
                        av = jnp.where(valid, av, _f32(0.0))
                        idxc = jnp.clip(local, 0, nbuf - 1)
                        dstmod_v[u, sl] = idxc
                        for r2 in range(16):
                            a = _bcast_lane(av, r2)
                            r = q * 16 + r2
                            for k2 in range(8):
                                slk = pl.ds(k2 * 16, 16)
                                gb[r, slk] = gb[r, slk] * a
                        return c2

                    lax.fori_loop(0, 8, _mul, 0)
                    pltpu.async_copy(gb, out_sh.at[dstmod_v.at[u]],
                                     semss[u], add=True)
                return carry

            lax.fori_loop(0, jcnt // 2, _s2, 0)
            for u in (0, 1):
                pltpu.make_async_copy(gbufs[u], out_sh.at[dstmod_v.at[u]],
                                      semss[u]).wait()
            plsc.subcore_barrier()
            off = 0
            for ln in wchunks:
                r0 = tid * rpt + off
                pltpu.sync_copy(out_sh.at[pl.ds(r0, ln)],
                                out_pd.at[core, cf, pl.ds(base + r0, ln)])
                off += ln


_sc_edge = functools.partial(
    pl.kernel,
    mesh=plsc.VectorSubcoreMesh(core_axis_name="c", subcore_axis_name="s"),
    out_type=jax.ShapeDtypeStruct((NCORE, NFC, NPAD, F), _f32),
    scratch_types=[
        pltpu.VMEM((N,), _f32),
        pltpu.VMEM((DEN,), _f32),
        pltpu.VMEM((ROWS1, CHUNK), jnp.int32),
        pltpu.VMEM((ROWS1, CHUNK), jnp.int32),
        pltpu.VMEM((ROWS1, CHUNK), _f32),
        pltpu.VMEM((16, F), _f32),
        pltpu.VMEM((640,), _f32),
        pltpu.VMEM((CHUNK, F), _f32),
        pltpu.VMEM((CHUNK, F), _f32),
        pltpu.VMEM((2, CHUNK), jnp.int32),
        pltpu.VMEM_SHARED((DEN,), _f32),
        pltpu.VMEM_SHARED((NBUF, F), _f32),
        pltpu.SemaphoreType.DMA,
        pltpu.SemaphoreType.DMA,
        pltpu.SemaphoreType.DMA,
        pltpu.SemaphoreType.DMA,
        pltpu.SemaphoreType.DMA,
    ],
    compiler_params=pltpu.CompilerParams(needs_layout_passes=False),
)(_sc_edge_body)


def kernel(x, edge_index, W1, al1, ar1, b1, W2, al2, ar2, b2,
           fc1_W, fc1_b, fc2_W, fc2_b):
    ei = edge_index.astype(jnp.int32)
    src = jnp.pad(ei[0], (0, EPAD - E)).reshape(NT, ROWS1, CHUNK)
    dst = jnp.pad(ei[1], (0, EPAD - E)).reshape(NT, ROWS1, CHUNK)

    h1c0, h1c1, h1c2, h1c3, el1, er1 = _p1(
        x, W1, al1.reshape(512, 1), ar1.reshape(512, 1))
    pd1 = _sc_edge(h1c0, h1c1, h1c2, h1c3,
                   el1.reshape(N), er1.reshape(N), src, dst)

    h2c0, h2c1, h2c2, h2c3, el2, er2 = _p3(
        pd1, b1.reshape(1, 512), W2, al2.reshape(512, 1), ar2.reshape(512, 1))
    pd2 = _sc_edge(h2c0, h2c1, h2c2, h2c3,
                   el2.reshape(N), er2.reshape(N), src, dst)

    fc2_Wp = jnp.pad(fc2_W, ((0, 0), (0, F - 2)))
    fc2_bp = jnp.pad(fc2_b, (0, F - 2)).reshape(1, F)
    (res,) = _p5(pd2, b2.reshape(NFC, F), fc1_W, fc1_b.reshape(1, 512),
                 fc2_Wp, fc2_bp)
    return res[0, :2]


# speedup vs baseline: 2.6431x; 1.5088x over previous
import functools

import jax
import jax.numpy as jnp
from jax import lax
from jax.experimental import pallas as pl
from jax.experimental.pallas import tpu as pltpu
from jax.experimental.pallas import tpu_sc as plsc

N = 10000
E = 160000
EPAD = 163840
NT = 16
NCORE = 2
CHUNK = 128
ROWS1 = 80
ROWS2 = 40
F = 128
NFC = 4
NPAD = 10240
DEN = 10240
NBUF = 5120
C0R = 56
MBLK = 1000
NMB = 10

ZONES = ((0, NBUF, 320, (128, 128, 64)),
         (NBUF, NBUF, 320, (128, 128, 64)))

_f32 = jnp.float32


def _bcast_lane(v16, lane):
    idx = jnp.full((16, 1), lane, dtype=jnp.int32)
    dnums = lax.GatherDimensionNumbers(
        offset_dims=(), collapsed_slice_dims=(0,), start_index_map=(0,))
    return lax.gather(v16, idx, dnums, (1,),
                      mode=lax.GatherScatterMode.PROMISE_IN_BOUNDS)


def _p1_body(x_ref, w_ref, al_ref, ar_ref, h0, h1, h2, h3, el_ref, er_ref):
    h = jnp.dot(x_ref[...], w_ref[...], preferred_element_type=_f32)
    h0[...] = h[:, 0:128]
    h1[...] = h[:, 128:256]
    h2[...] = h[:, 256:384]
    h3[...] = h[:, 384:512]
    el_ref[...] = jnp.dot(h, al_ref[...], preferred_element_type=_f32)
    er_ref[...] = jnp.dot(h, ar_ref[...], preferred_element_type=_f32)


_p1 = pl.pallas_call(
    _p1_body,
    grid=(NMB,),
    in_specs=[
        pl.BlockSpec((MBLK, 256), lambda m: (m, 0)),
        pl.BlockSpec((256, 512), lambda m: (0, 0)),
        pl.BlockSpec((512, 1), lambda m: (0, 0)),
        pl.BlockSpec((512, 1), lambda m: (0, 0)),
    ],
    out_specs=[pl.BlockSpec((MBLK, F), lambda m: (m, 0))] * 4
    + [pl.BlockSpec((MBLK, 1), lambda m: (m, 0))] * 2,
    out_shape=[jax.ShapeDtypeStruct((N, F), _f32)] * 4
    + [jax.ShapeDtypeStruct((N, 1), _f32)] * 2,
)


def _p3_body(pd_ref, b_ref, w_ref, al_ref, ar_ref,
             h0, h1, h2, h3, el_ref, er_ref, acc_ref):
    c = pl.program_id(1)
    xin = jnp.maximum(pd_ref[0, 0] + pd_ref[1, 0] + b_ref[...], 0.0)
    part = jnp.dot(xin, w_ref[...], preferred_element_type=_f32)

    @pl.when(c == 0)
    def _():
        acc_ref[...] = part

    @pl.when(c > 0)
    def _():
        acc_ref[...] = acc_ref[...] + part

    @pl.when(c == NFC - 1)
    def _():
        h = acc_ref[...]
        h0[...] = h[:, 0:128]
        h1[...] = h[:, 128:256]
        h2[...] = h[:, 256:384]
        h3[...] = h[:, 384:512]
        el_ref[...] = jnp.dot(h, al_ref[...], preferred_element_type=_f32)
        er_ref[...] = jnp.dot(h, ar_ref[...], preferred_element_type=_f32)


_p3 = pl.pallas_call(
    _p3_body,
    grid=(NMB, NFC),
    in_specs=[
        pl.BlockSpec((2, 1, MBLK, F), lambda m, c: (0, c, m, 0)),
        pl.BlockSpec((1, F), lambda m, c: (0, c)),
        pl.BlockSpec((F, 512), lambda m, c: (c, 0)),
        pl.BlockSpec((512, 1), lambda m, c: (0, 0)),
        pl.BlockSpec((512, 1), lambda m, c: (0, 0)),
    ],
    out_specs=[pl.BlockSpec((MBLK, F), lambda m, c: (m, 0))] * 4
    + [pl.BlockSpec((MBLK, 1), lambda m, c: (m, 0))] * 2,
    out_shape=[jax.ShapeDtypeStruct((N, F), _f32)] * 4
    + [jax.ShapeDtypeStruct((N, 1), _f32)] * 2,
    scratch_shapes=[pltpu.VMEM((MBLK, 512), _f32)],
    compiler_params=pltpu.CompilerParams(
        dimension_semantics=("parallel", "arbitrary")),
)


def _p5_body(pd_ref, b_ref, f1w_ref, f1b_ref, f2w_ref, f2b_ref,
             out_ref, acc_ref):
    m = pl.program_id(0)
    xin = jnp.maximum(pd_ref[0] + pd_ref[1] + b_ref[...][:, None, :], 0.0)
    cs = jnp.sum(xin, axis=1)

    @pl.when(m == 0)
    def _():
        acc_ref[...] = cs

    @pl.when(m > 0)
    def _():
        acc_ref[...] = acc_ref[...] + cs

    @pl.when(m == NMB - 1)
    def _():
        hh = f1b_ref[...]
        for c in range(NFC):
            hg_c = acc_ref[c:c + 1, :] * _f32(1.0 / N)
            hh = hh + jnp.dot(hg_c, f1w_ref[c * F:(c + 1) * F, :],
                              preferred_element_type=_f32)
        hh = jnp.maximum(hh, 0.0)
        out_ref[...] = jnp.dot(hh, f2w_ref[...],
                               preferred_element_type=_f32) + f2b_ref[...]


_p5 = pl.pallas_call(
    _p5_body,
    grid=(NMB,),
    in_specs=[
        pl.BlockSpec((2, NFC, MBLK, F), lambda m: (0, 0, m, 0)),
        pl.BlockSpec((NFC, F), lambda m: (0, 0)),
        pl.BlockSpec((512, 512), lambda m: (0, 0)),
        pl.BlockSpec((1, 512), lambda m: (0, 0)),
        pl.BlockSpec((512, F), lambda m: (0, 0)),
        pl.BlockSpec((1, F), lambda m: (0, 0)),
    ],
    out_specs=[pl.BlockSpec((1, F), lambda m: (0, 0))],
    out_shape=[jax.ShapeDtypeStruct((1, F), _f32)],
    scratch_shapes=[pltpu.VMEM((NFC, F), _f32)],
    compiler_params=pltpu.CompilerParams(
        dimension_semantics=("arbitrary",)),
)


def _sc_edge_body(h0, h1, h2, h3, el_h, er_h, src_h, dst_h, out_pd,
                  el_v, erden_v, src_v, dst_v, ex_v, zbuf, zd_v,
                  gbuf0, gbuf1, dstmod_v, den_sh, out_sh,
                  sem, semg0, semg1, sems0, sems1):
    core = lax.axis_index("c")
    tid = lax.axis_index("s")
    hcs = (h0, h1, h2, h3)
    gbufs = (gbuf0, gbuf1)
    semgs = (semg0, semg1)
    semss = (sems0, sems1)

    def _zb(i, carry):
        for q in range(8):
            zbuf[i, pl.ds(q * 16, 16)] = jnp.zeros((16,), _f32)
        return carry

    lax.fori_loop(0, 16, _zb, 0)

    def _zd(i, carry):
        zd_v[pl.ds(i * 16, 16)] = jnp.zeros((16,), _f32)
        return carry

    lax.fori_loop(0, 40, _zd, 0)

    pltpu.sync_copy(el_h, el_v)
    pltpu.sync_copy(er_h, erden_v.at[pl.ds(0, N)])
    pltpu.sync_copy(src_h.at[tid], src_v)
    pltpu.sync_copy(dst_h.at[tid], dst_v)
    pltpu.sync_copy(zd_v, den_sh.at[pl.ds(tid * 640, 640)])
    plsc.subcore_barrier()

    def _s1(j, carry):
        for q in range(8):
            sl = pl.ds(q * 16, 16)
            sv = src_v[j, sl]
            dv = dst_v[j, sl]
            es = plsc.load_gather(el_v, [sv])
            ed = plsc.load_gather(erden_v, [dv])
            z = es + ed
            e = jnp.where(z >= 0.0, z, z * _f32(0.2))
            ex = jnp.exp(e)
            ex = jnp.where(sv < N, ex, _f32(0.0))
            ex_v[j, sl] = ex
        return carry

    lax.fori_loop(0, ROWS1, _s1, 0)

    for blk in range(ROWS1 // 16):
        def _fire(i, carry, blk=blk):
            j = blk * 16 + i
            pltpu.async_copy(ex_v.at[j], den_sh.at[dst_v.at[j]], sem,
                             add=True)
            return carry

        lax.fori_loop(0, 16, _fire, 0)

        def _drain(i, carry, blk=blk):
            j = blk * 16 + i
            pltpu.make_async_copy(ex_v.at[j], den_sh.at[dst_v.at[j]],
                                  sem).wait()
            return carry

        lax.fori_loop(0, 16, _drain, 0)

    plsc.subcore_barrier()
    pltpu.sync_copy(den_sh, erden_v)

    jlo = core * C0R
    jcnt = C0R - core * (2 * C0R - ROWS1)

    def _s1b(j, carry):
        jabs = jlo + j
        for q in range(8):
            sl = pl.ds(q * 16, 16)
            ex = ex_v[jabs, sl]
            dv = dst_v[jabs, sl]
            d = plsc.load_gather(erden_v, [dv])
            ex_v[jabs, sl] = ex / jnp.maximum(d, _f32(1e-30))
            carry = carry + plsc.all_reduce_population_count(dv < NBUF)
        return carry

    na_vec = lax.fori_loop(0, jcnt, _s1b, jnp.zeros((16,), jnp.int32))
    na = jnp.max(na_vec)
    cnt_a = ((na + 255) // 256) * 2
    s0_b = (na // 256) * 2
    cnt_b = jcnt - s0_b

    for cf in range(NFC):
        hc = hcs[cf]
        for base, nbuf, rpt, wchunks, s0, cnt in (
                ZONES[0] + (0, cnt_a), ZONES[1] + (s0_b, cnt_b)):
            plsc.subcore_barrier()
            off = 0
            for ln in wchunks:
                while ln > 0:
                    zl = min(ln, 16)
                    pltpu.sync_copy(zbuf.at[pl.ds(0, zl)],
                                    out_sh.at[pl.ds(tid * rpt + off, zl)])
                    off += zl
                    ln -= zl
            plsc.subcore_barrier()

            @pl.when(cnt > 0)
            def _(hc=hc, s0=s0):
                pltpu.async_copy(hc.at[src_v.at[jlo + s0]], gbufs[0],
                                 semgs[0])

            def _s2(jj, carry, hc=hc, base=base, nbuf=nbuf, s0=s0, cnt=cnt):
                for u in range(2):
                    j = jj * 2 + u
                    un = u ^ 1
                    jabs = jlo + s0 + j

                    @pl.when((j >= 1) & (j + 1 < cnt))
                    def _(un=un):
                        pltpu.make_async_copy(
                            gbufs[un], out_sh.at[dstmod_v.at[un]],
                            semss[un]).wait()

                    @pl.when(j + 1 < cnt)
                    def _(jabs=jabs, un=un):
                        pltpu.async_copy(hc.at[src_v.at[jabs + 1]],
                                         gbufs[un], semgs[un])

                    gb = gbufs[u]
                    pltpu.make_async_copy(
                        hc.at[src_v.at[jabs]], gb, semgs[u]).wait()

                    def _mul(q, c2, jabs=jabs, u=u, gb=gb):
                        sl = pl.ds(q * 16, 16)
                        av = ex_v[jabs, sl]
                        dv = dst_v[jabs, sl]
                        local = dv - base
                        valid = (local >= 0) & (local < nbuf)
                        av = jnp.where(valid, av, _f32(0.0))
                        idxc = jnp.clip(local, 0, nbuf - 1)
                        dstmod_v[u, sl] = idxc
                        for r2 in range(16):
                            a = _bcast_lane(av, r2)
                            r = q * 16 + r2
                            for k2 in range(8):
                                slk = pl.ds(k2 * 16, 16)
                                gb[r, slk] = gb[r, slk] * a
                        return c2

                    lax.fori_loop(0, 8, _mul, 0)
                    pltpu.async_copy(gb, out_sh.at[dstmod_v.at[u]],
                                     semss[u], add=True)
                return carry

            lax.fori_loop(0, cnt // 2, _s2, 0)
            for u in (0, 1):
                @pl.when(cnt > 0)
                def _(u=u):
                    pltpu.make_async_copy(gbufs[u],
                                          out_sh.at[dstmod_v.at[u]],
                                          semss[u]).wait()
            plsc.subcore_barrier()
            off = 0
            for ln in wchunks:
                r0 = tid * rpt + off
                pltpu.sync_copy(out_sh.at[pl.ds(r0, ln)],
                                out_pd.at[core, cf, pl.ds(base + r0, ln)])
                off += ln


_sc_edge = functools.partial(
    pl.kernel,
    mesh=plsc.VectorSubcoreMesh(core_axis_name="c", subcore_axis_name="s"),
    out_type=jax.ShapeDtypeStruct((NCORE, NFC, NPAD, F), _f32),
    scratch_types=[
        pltpu.VMEM((DEN,), _f32),
        pltpu.VMEM((DEN,), _f32),
        pltpu.VMEM((ROWS1, CHUNK), jnp.int32),
        pltpu.VMEM((ROWS1, CHUNK), jnp.int32),
        pltpu.VMEM((ROWS1, CHUNK), _f32),
        pltpu.VMEM((16, F), _f32),
        pltpu.VMEM((640,), _f32),
        pltpu.VMEM((CHUNK, F), _f32),
        pltpu.VMEM((CHUNK, F), _f32),
        pltpu.VMEM((2, CHUNK), jnp.int32),
        pltpu.VMEM_SHARED((DEN,), _f32),
        pltpu.VMEM_SHARED((NBUF, F), _f32),
        pltpu.SemaphoreType.DMA,
        pltpu.SemaphoreType.DMA,
        pltpu.SemaphoreType.DMA,
        pltpu.SemaphoreType.DMA,
        pltpu.SemaphoreType.DMA,
    ],
    compiler_params=pltpu.CompilerParams(needs_layout_passes=False),
)(_sc_edge_body)


def kernel(x, edge_index, W1, al1, ar1, b1, W2, al2, ar2, b2,
           fc1_W, fc1_b, fc2_W, fc2_b):
    ei = edge_index.astype(jnp.int32)
    srcp = jnp.pad(ei[0], (0, EPAD - E), constant_values=N)
    dstp = jnp.pad(ei[1], (0, EPAD - E), constant_values=N - 1)
    order = jnp.argsort((dstp >= NBUF).astype(jnp.int32), stable=True)
    src = srcp[order].reshape(NT, ROWS1, CHUNK)
    dst = dstp[order].reshape(NT, ROWS1, CHUNK)

    h1c0, h1c1, h1c2, h1c3, el1, er1 = _p1(
        x, W1, al1.reshape(512, 1), ar1.reshape(512, 1))
    pd1 = _sc_edge(h1c0, h1c1, h1c2, h1c3,
                   jnp.pad(el1.reshape(N), (0, DEN - N)),
                   er1.reshape(N), src, dst)

    h2c0, h2c1, h2c2, h2c3, el2, er2 = _p3(
        pd1, b1.reshape(1, 512), W2, al2.reshape(512, 1), ar2.reshape(512, 1))
    pd2 = _sc_edge(h2c0, h2c1, h2c2, h2c3,
                   jnp.pad(el2.reshape(N), (0, DEN - N)),
                   er2.reshape(N), src, dst)

    fc2_Wp = jnp.pad(fc2_W, ((0, 0), (0, F - 2)))
    fc2_bp = jnp.pad(fc2_b, (0, F - 2)).reshape(1, F)
    (res,) = _p5(pd2, b2.reshape(NFC, F), fc1_W, fc1_b.reshape(1, 512),
                 fc2_Wp, fc2_bp)
    return res[0, :2]
